# baseline (speedup 1.0000x reference)
import numpy as np
import jax
import jax.numpy as jnp
from jax.experimental import pallas as pl
from jax.experimental.pallas import tpu as pltpu

S, D, H, HD, BS = 2048, 2048, 16, 128, 128
NB = S // BS
NPAIR = NB * (NB + 1) // 2
SCALE = 1.0 / np.sqrt(float(HD))
NEG = -1e9

_PQ = np.concatenate([np.full(q + 1, q, np.int32) for q in range(NB)])
_PK = np.concatenate([np.arange(q + 1, dtype=np.int32) for q in range(NB)])


def _qkv_kernel(x_ref, w_ref, cos_ref, sin_ref, q_ref, k_ref, v_ref,
                qp_ref, kp_ref):
    x = x_ref[...]
    qkv = jnp.dot(x, w_ref[...], preferred_element_type=jnp.float32)
    cos = cos_ref[...]
    sin = sin_ref[...]

    def rope(t):
        outs = []
        for h in range(H):
            th = t[:, h * HD:(h + 1) * HD]
            rot = jnp.concatenate([-th[:, HD // 2:], th[:, :HD // 2]], axis=1)
            outs.append(th * cos + rot * sin)
        return jnp.concatenate(outs, axis=1)

    q = rope(qkv[:, :D])
    k = rope(qkv[:, D:2 * D])
    q_ref[...] = q
    k_ref[...] = k
    v_ref[...] = qkv[:, 2 * D:]
    qp_ref[...] = jnp.mean(q, axis=0).reshape(1, 1, D)
    kp_ref[...] = jnp.mean(k, axis=0).reshape(1, 1, D)


def _gate_kernel(qp_ref, kp_ref, gate_ref):
    qp = qp_ref[:, 0, :]
    kp = kp_ref[:, 0, :]
    s = jax.lax.dot_general(qp, kp, (((1,), (1,)), ((), ())),
                            preferred_element_type=jnp.float32) * SCALE
    r = jax.lax.broadcasted_iota(jnp.int32, (NB, NB), 0)
    c = jax.lax.broadcasted_iota(jnp.int32, (NB, NB), 1)
    on = (jax.nn.sigmoid(s) >= 0.5) & (r >= c) | (r == c)
    gate_ref[...] = on.astype(jnp.int32).reshape(1, NB, NB)


def _attn_kernel(pq_ref, pk_ref, kidx_ref, gon_ref, q_ref, k_ref, v_ref,
                 o_ref, m_ref, l_ref, acc_ref):
    h = pl.program_id(0)
    p = pl.program_id(1)
    qb = pq_ref[p]
    kb = pk_ref[p]

    @pl.when(kb == 0)
    def _init():
        m_ref[...] = jnp.full_like(m_ref, -1e30)
        l_ref[...] = jnp.zeros_like(l_ref)
        acc_ref[...] = jnp.zeros_like(acc_ref)

    @pl.when(gon_ref[h * NPAIR + p] > 0)
    def _compute():
        q = q_ref[...]
        k = k_ref[...]
        s = jax.lax.dot_general(q, k, (((1,), (1,)), ((), ())),
                                preferred_element_type=jnp.float32) * SCALE
        r = jax.lax.broadcasted_iota(jnp.int32, (BS, BS), 0)
        c = jax.lax.broadcasted_iota(jnp.int32, (BS, BS), 1)
        s = jnp.where(jnp.logical_or(kb < qb, r >= c), s, NEG)
        m_prev = m_ref[...]
        m_new = jnp.maximum(m_prev, jnp.max(s, axis=1, keepdims=True))
        alpha = jnp.exp(m_prev - m_new)
        pmat = jnp.exp(s - m_new)
        l_ref[...] = l_ref[...] * alpha + jnp.sum(pmat, axis=1, keepdims=True)
        acc_ref[...] = acc_ref[...] * alpha + jnp.dot(
            pmat, v_ref[...], preferred_element_type=jnp.float32)
        m_ref[...] = m_new

    @pl.when(kb == qb)
    def _fin():
        o_ref[...] = acc_ref[...] / l_ref[...]


def _proj_kernel(x_ref, w_ref, o_ref):
    o_ref[...] = jnp.dot(x_ref[...], w_ref[...],
                         preferred_element_type=jnp.float32)


def kernel(hidden_states, cos, sin, Wq, Wk, Wv, Wo):
    x = hidden_states[0]
    cosb = cos[0]
    sinb = sin[0]
    wcat = jnp.concatenate([Wq.T, Wk.T, Wv.T], axis=1)

    q, k, v, qp, kp = pl.pallas_call(
        _qkv_kernel,
        grid=(NB,),
        in_specs=[
            pl.BlockSpec((BS, D), lambda i: (i, 0)),
            pl.BlockSpec((D, 3 * D), lambda i: (0, 0)),
            pl.BlockSpec((BS, HD), lambda i: (i, 0)),
            pl.BlockSpec((BS, HD), lambda i: (i, 0)),
        ],
        out_specs=[
            pl.BlockSpec((BS, D), lambda i: (i, 0)),
            pl.BlockSpec((BS, D), lambda i: (i, 0)),
            pl.BlockSpec((BS, D), lambda i: (i, 0)),
            pl.BlockSpec((1, 1, D), lambda i: (i, 0, 0)),
            pl.BlockSpec((1, 1, D), lambda i: (i, 0, 0)),
        ],
        out_shape=[
            jax.ShapeDtypeStruct((S, D), jnp.float32),
            jax.ShapeDtypeStruct((S, D), jnp.float32),
            jax.ShapeDtypeStruct((S, D), jnp.float32),
            jax.ShapeDtypeStruct((NB, 1, D), jnp.float32),
            jax.ShapeDtypeStruct((NB, 1, D), jnp.float32),
        ],
    )(x, wcat, cosb, sinb)

    gate = pl.pallas_call(
        _gate_kernel,
        grid=(H,),
        in_specs=[
            pl.BlockSpec((NB, 1, HD), lambda h: (0, 0, h)),
            pl.BlockSpec((NB, 1, HD), lambda h: (0, 0, h)),
        ],
        out_specs=pl.BlockSpec((1, NB, NB), lambda h: (h, 0, 0)),
        out_shape=jax.ShapeDtypeStruct((H, NB, NB), jnp.int32),
    )(qp, kp)

    idx = jnp.arange(NB, dtype=jnp.int32)
    masked = jnp.where(gate > 0, idx[None, None, :], -1)
    run = jax.lax.cummax(masked, axis=2)
    first = jnp.argmax(gate > 0, axis=2).astype(jnp.int32)
    kmap = jnp.where(run < 0, first[:, :, None], run)
    pq = jnp.asarray(_PQ)
    pk = jnp.asarray(_PK)
    kidx = kmap[:, pq, pk].reshape(-1).astype(jnp.int32)
    gon = gate[:, pq, pk].reshape(-1).astype(jnp.int32)

    o = pl.pallas_call(
        _attn_kernel,
        grid_spec=pltpu.PrefetchScalarGridSpec(
            num_scalar_prefetch=4,
            grid=(H, NPAIR),
            in_specs=[
                pl.BlockSpec((BS, HD),
                             lambda h, p, pq, pk, ki, go: (pq[p], h)),
                pl.BlockSpec((BS, HD),
                             lambda h, p, pq, pk, ki, go: (ki[h * NPAIR + p], h)),
                pl.BlockSpec((BS, HD),
                             lambda h, p, pq, pk, ki, go: (ki[h * NPAIR + p], h)),
            ],
            out_specs=pl.BlockSpec((BS, HD),
                                   lambda h, p, pq, pk, ki, go: (pq[p], h)),
            scratch_shapes=[
                pltpu.VMEM((BS, 1), jnp.float32),
                pltpu.VMEM((BS, 1), jnp.float32),
                pltpu.VMEM((BS, HD), jnp.float32),
            ],
        ),
        out_shape=jax.ShapeDtypeStruct((S, D), jnp.float32),
    )(pq, pk, kidx, gon, q, k, v)

    out = pl.pallas_call(
        _proj_kernel,
        grid=(NB,),
        in_specs=[
            pl.BlockSpec((BS, D), lambda i: (i, 0)),
            pl.BlockSpec((D, D), lambda i: (0, 0)),
        ],
        out_specs=pl.BlockSpec((BS, D), lambda i: (i, 0)),
        out_shape=jax.ShapeDtypeStruct((S, D), jnp.float32),
    )(o, Wo.T)

    return out[None]


# speedup vs baseline: 2.8708x; 2.8708x over previous
import numpy as np
import jax
import jax.numpy as jnp
from jax.experimental import pallas as pl
from jax.experimental.pallas import tpu as pltpu

S, D, H, HD, BS = 2048, 2048, 16, 128, 128
NB = S // BS
SCALE = 1.0 / np.sqrt(float(HD))
NEG = -1e9

_E = np.kron(np.eye(NB, dtype=np.float32), np.ones((1, BS), np.float32))

_TDIMS = (((1,), (1,)), ((), ()))


def _qkv_kernel(x_ref, wq_ref, wk_ref, wv_ref, cos_ref, sin_ref,
                q_ref, k_ref, v_ref, qp_ref, kp_ref):
    x = x_ref[...]
    q = jax.lax.dot_general(x, wq_ref[...], _TDIMS,
                            preferred_element_type=jnp.float32)
    k = jax.lax.dot_general(x, wk_ref[...], _TDIMS,
                            preferred_element_type=jnp.float32)
    v = jax.lax.dot_general(x.astype(jnp.bfloat16), wv_ref[...], _TDIMS,
                            preferred_element_type=jnp.float32)
    cos = cos_ref[...]
    sin = sin_ref[...]

    def rope(t):
        outs = []
        for h in range(H):
            th = t[:, h * HD:(h + 1) * HD]
            rot = jnp.concatenate([-th[:, HD // 2:], th[:, :HD // 2]], axis=1)
            outs.append(th * cos + rot * sin)
        return jnp.concatenate(outs, axis=1)

    q = rope(q)
    k = rope(k)
    q_ref[...] = q.astype(jnp.bfloat16)
    k_ref[...] = k.astype(jnp.bfloat16)
    v_ref[...] = v.astype(jnp.bfloat16)
    qp_ref[...] = jnp.mean(q, axis=0).reshape(1, 1, D)
    kp_ref[...] = jnp.mean(k, axis=0).reshape(1, 1, D)


def _attn_kernel(q_ref, k_ref, v_ref, qp_ref, kp_ref, e_ref, o_ref):
    qb = pl.program_id(1)
    qprow = qp_ref[0]
    kp = kp_ref[:, 0, :]
    srow = jax.lax.dot_general(qprow, kp, _TDIMS,
                               preferred_element_type=jnp.float32) * SCALE
    j = jax.lax.broadcasted_iota(jnp.int32, (1, NB), 1)
    bits = ((jax.nn.sigmoid(srow) >= 0.5) & (j <= qb)) | (j == qb)
    wide = jnp.dot(bits.astype(jnp.float32), e_ref[...],
                   preferred_element_type=jnp.float32)

    q = q_ref[...]
    s = jax.lax.dot_general(q, k_ref[...], _TDIMS,
                            preferred_element_type=jnp.float32) * SCALE
    r = qb * BS + jax.lax.broadcasted_iota(jnp.int32, (BS, S), 0)
    c = jax.lax.broadcasted_iota(jnp.int32, (BS, S), 1)
    allowed = (wide > 0.5) & (c <= r)
    s = jnp.where(allowed, s, NEG)
    m = jnp.max(s, axis=1, keepdims=True)
    p = jnp.exp(s - m)
    l = jnp.sum(p, axis=1, keepdims=True)
    o = jax.lax.dot_general(p.astype(jnp.bfloat16), v_ref[...],
                            (((1,), (0,)), ((), ())),
                            preferred_element_type=jnp.float32)
    o_ref[...] = (o / l).astype(jnp.bfloat16)


def _proj_kernel(x_ref, w_ref, o_ref):
    o_ref[...] = jax.lax.dot_general(x_ref[...], w_ref[...], _TDIMS,
                                     preferred_element_type=jnp.float32)


def kernel(hidden_states, cos, sin, Wq, Wk, Wv, Wo):
    x = hidden_states[0]
    cosb = cos[0]
    sinb = sin[0]

    q, k, v, qp, kp = pl.pallas_call(
        _qkv_kernel,
        grid=(NB,),
        in_specs=[
            pl.BlockSpec((BS, D), lambda i: (i, 0)),
            pl.BlockSpec((D, D), lambda i: (0, 0)),
            pl.BlockSpec((D, D), lambda i: (0, 0)),
            pl.BlockSpec((D, D), lambda i: (0, 0)),
            pl.BlockSpec((BS, HD), lambda i: (i, 0)),
            pl.BlockSpec((BS, HD), lambda i: (i, 0)),
        ],
        out_specs=[
            pl.BlockSpec((BS, D), lambda i: (i, 0)),
            pl.BlockSpec((BS, D), lambda i: (i, 0)),
            pl.BlockSpec((BS, D), lambda i: (i, 0)),
            pl.BlockSpec((1, 1, D), lambda i: (i, 0, 0)),
            pl.BlockSpec((1, 1, D), lambda i: (i, 0, 0)),
        ],
        out_shape=[
            jax.ShapeDtypeStruct((S, D), jnp.bfloat16),
            jax.ShapeDtypeStruct((S, D), jnp.bfloat16),
            jax.ShapeDtypeStruct((S, D), jnp.bfloat16),
            jax.ShapeDtypeStruct((NB, 1, D), jnp.float32),
            jax.ShapeDtypeStruct((NB, 1, D), jnp.float32),
        ],
    )(x, Wq, Wk, Wv.astype(jnp.bfloat16), cosb, sinb)

    o = pl.pallas_call(
        _attn_kernel,
        grid=(H, NB),
        in_specs=[
            pl.BlockSpec((BS, HD), lambda h, i: (i, h)),
            pl.BlockSpec((S, HD), lambda h, i: (0, h)),
            pl.BlockSpec((S, HD), lambda h, i: (0, h)),
            pl.BlockSpec((1, 1, HD), lambda h, i: (i, 0, h)),
            pl.BlockSpec((NB, 1, HD), lambda h, i: (0, 0, h)),
            pl.BlockSpec((NB, S), lambda h, i: (0, 0)),
        ],
        out_specs=pl.BlockSpec((BS, HD), lambda h, i: (i, h)),
        out_shape=jax.ShapeDtypeStruct((S, D), jnp.bfloat16),
    )(q, k, v, qp, kp, jnp.asarray(_E))

    out = pl.pallas_call(
        _proj_kernel,
        grid=(NB,),
        in_specs=[
            pl.BlockSpec((BS, D), lambda i: (i, 0)),
            pl.BlockSpec((D, D), lambda i: (0, 0)),
        ],
        out_specs=pl.BlockSpec((BS, D), lambda i: (i, 0)),
        out_shape=jax.ShapeDtypeStruct((S, D), jnp.float32),
    )(o, Wo.astype(jnp.bfloat16))

    return out[None]


# speedup vs baseline: 4.3661x; 1.5209x over previous
import numpy as np
import jax
import jax.numpy as jnp
from jax.experimental import pallas as pl
from jax.experimental.pallas import tpu as pltpu

S, D, H, HD, BS = 2048, 2048, 16, 128, 128
NB = S // BS
RT = 256
NRT = S // RT
GPT = RT // BS
SCALE = 1.0 / np.sqrt(float(HD))
NEG = -1e9

_E = np.kron(np.eye(NB, dtype=np.float32), np.ones((1, BS), np.float32))

_TDIMS = (((1,), (1,)), ((), ()))


def _qkv_kernel(x_ref, wq_ref, wk_ref, wv_ref, cos_ref, sin_ref,
                q_ref, k_ref, v_ref, qp_ref, kp_ref):
    x = x_ref[...]
    q = jax.lax.dot_general(x, wq_ref[...], _TDIMS,
                            preferred_element_type=jnp.float32)
    k = jax.lax.dot_general(x, wk_ref[...], _TDIMS,
                            preferred_element_type=jnp.float32)
    v = jax.lax.dot_general(x.astype(jnp.bfloat16), wv_ref[...], _TDIMS,
                            preferred_element_type=jnp.float32)
    cos = cos_ref[...]
    sin = sin_ref[...]

    def rope(t):
        outs = []
        for h in range(H):
            th = t[:, h * HD:(h + 1) * HD]
            rot = jnp.concatenate([-th[:, HD // 2:], th[:, :HD // 2]], axis=1)
            outs.append(th * cos + rot * sin)
        return jnp.concatenate(outs, axis=1)

    q = rope(q)
    k = rope(k)
    q_ref[...] = q.astype(jnp.bfloat16)
    k_ref[...] = k.astype(jnp.bfloat16)
    v_ref[...] = v.astype(jnp.bfloat16)
    qp_ref[...] = jnp.concatenate(
        [jnp.mean(q[g * BS:(g + 1) * BS], axis=0).reshape(1, 1, D)
         for g in range(GPT)], axis=0)
    kp_ref[...] = jnp.concatenate(
        [jnp.mean(k[g * BS:(g + 1) * BS], axis=0).reshape(1, 1, D)
         for g in range(GPT)], axis=0)


def _attn_kernel(q_ref, k_ref, v_ref, qp_ref, kp_ref, e_ref, o_ref):
    qt = pl.program_id(1)
    qprows = qp_ref[:, 0, :]
    kp = kp_ref[:, 0, :]
    srow = jax.lax.dot_general(qprows, kp, _TDIMS,
                               preferred_element_type=jnp.float32) * SCALE
    g = jax.lax.broadcasted_iota(jnp.int32, (GPT, NB), 0)
    j = jax.lax.broadcasted_iota(jnp.int32, (GPT, NB), 1)
    qbg = GPT * qt + g
    bits = ((jax.nn.sigmoid(srow) >= 0.5) & (j <= qbg)) | (j == qbg)
    wide = jnp.dot(bits.astype(jnp.float32), e_ref[...],
                   preferred_element_type=jnp.float32)

    q = q_ref[...]
    s = jax.lax.dot_general(q, k_ref[...], _TDIMS,
                            preferred_element_type=jnp.float32) * SCALE
    r = qt * RT + jax.lax.broadcasted_iota(jnp.int32, (RT, S), 0)
    c = jax.lax.broadcasted_iota(jnp.int32, (RT, S), 1)
    rg = (jax.lax.broadcasted_iota(jnp.int32, (RT, 1), 0) // BS
          ).astype(jnp.float32)
    mask_f = wide[0:1, :] * (1.0 - rg) + wide[1:2, :] * rg
    allowed = (mask_f > 0.5) & (c <= r)
    s = jnp.where(allowed, s, NEG)
    m = jnp.max(s, axis=1, keepdims=True)
    p = jnp.exp(s - m)
    l = jnp.sum(p, axis=1, keepdims=True)
    o = jax.lax.dot_general(p.astype(jnp.bfloat16), v_ref[...],
                            (((1,), (0,)), ((), ())),
                            preferred_element_type=jnp.float32)
    o_ref[...] = (o / l).astype(jnp.bfloat16)


def _proj_kernel(x_ref, w_ref, o_ref):
    o_ref[...] = jax.lax.dot_general(x_ref[...], w_ref[...], _TDIMS,
                                     preferred_element_type=jnp.float32)


def kernel(hidden_states, cos, sin, Wq, Wk, Wv, Wo):
    x = hidden_states[0]
    cosb = cos[0]
    sinb = sin[0]

    q, k, v, qp, kp = pl.pallas_call(
        _qkv_kernel,
        grid=(NRT,),
        in_specs=[
            pl.BlockSpec((RT, D), lambda i: (i, 0)),
            pl.BlockSpec((D, D), lambda i: (0, 0)),
            pl.BlockSpec((D, D), lambda i: (0, 0)),
            pl.BlockSpec((D, D), lambda i: (0, 0)),
            pl.BlockSpec((RT, HD), lambda i: (i, 0)),
            pl.BlockSpec((RT, HD), lambda i: (i, 0)),
        ],
        out_specs=[
            pl.BlockSpec((RT, D), lambda i: (i, 0)),
            pl.BlockSpec((RT, D), lambda i: (i, 0)),
            pl.BlockSpec((RT, D), lambda i: (i, 0)),
            pl.BlockSpec((GPT, 1, D), lambda i: (i, 0, 0)),
            pl.BlockSpec((GPT, 1, D), lambda i: (i, 0, 0)),
        ],
        out_shape=[
            jax.ShapeDtypeStruct((S, D), jnp.bfloat16),
            jax.ShapeDtypeStruct((S, D), jnp.bfloat16),
            jax.ShapeDtypeStruct((S, D), jnp.bfloat16),
            jax.ShapeDtypeStruct((NB, 1, D), jnp.float32),
            jax.ShapeDtypeStruct((NB, 1, D), jnp.float32),
        ],
    )(x, Wq, Wk, Wv.astype(jnp.bfloat16), cosb, sinb)

    o = pl.pallas_call(
        _attn_kernel,
        grid=(H, NRT),
        in_specs=[
            pl.BlockSpec((RT, HD), lambda h, i: (i, h)),
            pl.BlockSpec((S, HD), lambda h, i: (0, h)),
            pl.BlockSpec((S, HD), lambda h, i: (0, h)),
            pl.BlockSpec((GPT, 1, HD), lambda h, i: (i, 0, h)),
            pl.BlockSpec((NB, 1, HD), lambda h, i: (0, 0, h)),
            pl.BlockSpec((NB, S), lambda h, i: (0, 0)),
        ],
        out_specs=pl.BlockSpec((RT, HD), lambda h, i: (i, h)),
        out_shape=jax.ShapeDtypeStruct((S, D), jnp.bfloat16),
    )(q, k, v, qp, kp, jnp.asarray(_E))

    out = pl.pallas_call(
        _proj_kernel,
        grid=(NRT,),
        in_specs=[
            pl.BlockSpec((RT, D), lambda i: (i, 0)),
            pl.BlockSpec((D, D), lambda i: (0, 0)),
        ],
        out_specs=pl.BlockSpec((RT, D), lambda i: (i, 0)),
        out_shape=jax.ShapeDtypeStruct((S, D), jnp.float32),
    )(o, Wo.astype(jnp.bfloat16))

    return out[None]
